# baseline (speedup 1.0000x reference)
import jax
import jax.numpy as jnp
from jax import lax
from jax.experimental import pallas as pl
from jax.experimental.pallas import tpu as pltpu

_B, _N, _INF, _OUTF = 4, 1024, 256, 256
_K = 256
_T = _N // _K


def _gat_body(inp_ref, adj_ref, w_ref, a_ref, out_ref,
              hp_s, m_s, abot_s, vals1_s, s_s, hlast_s):
    s = pl.program_id(0)

    @pl.when(s < _T)
    def _stream():
        xf = inp_ref[...].reshape(_B * _K, _INF)
        hh = jnp.dot(xf, w_ref[...], preferred_element_type=jnp.float32)
        kin = lax.broadcasted_iota(jnp.int32, (_B * _K, 1), 0) % _K
        hh = jnp.where((kin == 0) & (s == 0), 0.0, hh)
        hT = lax.dot_general(w_ref[...], xf, (((0,), (1,)), ((), ())),
                             preferred_element_type=jnp.float32)
        m_s[pl.ds(s * _K, _K), :] = (adj_ref[...] > 0).astype(jnp.bfloat16)
        a_blk = a_ref[...]
        atop_t = a_blk[:_OUTF, :]
        abot_s[:, pl.ds(s * _K, _K)] = a_blk[_OUTF:, :]

        kk = lax.broadcasted_iota(jnp.int32, (_K, 1), 0)
        for b in range(_B):
            hb = lax.slice(hh, (b * _K, 0), ((b + 1) * _K, _OUTF))
            carry = jnp.where(s == 0, 0.0, hlast_s[pl.ds(b, 1), :])
            hpb = pltpu.roll(hb, 1, 0)
            hpb = jnp.where(kk == 0, carry, hpb)
            hlast_s[pl.ds(b, 1), :] = lax.slice(hb, (_K - 1, 0),
                                                (_K, _OUTF))
            hp_s[pl.ds(b * _N + s * _K, _K), :] = hpb.astype(jnp.bfloat16)
            sprev = jnp.where(s == 0, 0.0, s_s[pl.ds(b, 1), :])
            s_s[pl.ds(b, 1), :] = sprev + jnp.sum(hb, axis=0, keepdims=True)
            hT_b = lax.slice(hT, (0, b * _K), (_OUTF, (b + 1) * _K))
            v1 = jnp.sum(hT_b * atop_t, axis=0, keepdims=True)
            vals1_s[pl.ds(b, 1), pl.ds(s * _K, _K)] = v1

    @pl.when(s >= _T)
    def _finalize():
        b = s - _T
        hp = hp_s[pl.ds(b * _N, _N), :]
        gT = lax.dot_general(hp, m_s[...], (((0,), (0,)), ((), ())),
                             preferred_element_type=jnp.float32)
        vr = (vals1_s[pl.ds(b, 1), :]
              + jnp.sum(gT * abot_s[...], axis=0, keepdims=True))
        lane = lax.broadcasted_iota(jnp.int32, (1, _N), 1)
        vr = jnp.where(lane == 0, 0.0, vr)
        sr = s_s[pl.ds(b, 1), :]
        o = lax.dot_general(vr, sr, (((0,), (0,)), ((), ())),
                            preferred_element_type=jnp.float32)
        out_ref[0] = jnp.maximum(o, 0.2 * o)


def kernel(inp, adj, W, a):
    return pl.pallas_call(
        _gat_body,
        grid=(_T + _B,),
        in_specs=[
            pl.BlockSpec((_B, _K, _INF),
                         lambda s: (0, jnp.minimum(s, _T - 1), 0)),
            pl.BlockSpec((_K, _N),
                         lambda s: (jnp.minimum(s, _T - 1), 0)),
            pl.BlockSpec((_INF, _OUTF), lambda s: (0, 0)),
            pl.BlockSpec((2 * _OUTF, _K),
                         lambda s: (0, jnp.minimum(s, _T - 1))),
        ],
        out_specs=pl.BlockSpec((1, _N, _OUTF),
                               lambda s: (jnp.maximum(s - _T, 0), 0, 0)),
        out_shape=jax.ShapeDtypeStruct((_B, _N, _OUTF), jnp.float32),
        scratch_shapes=[
            pltpu.VMEM((_B * _N, _OUTF), jnp.bfloat16),
            pltpu.VMEM((_N, _N), jnp.bfloat16),
            pltpu.VMEM((_OUTF, _N), jnp.float32),
            pltpu.VMEM((8, _N), jnp.float32),
            pltpu.VMEM((8, _OUTF), jnp.float32),
            pltpu.VMEM((8, _OUTF), jnp.float32),
        ],
        compiler_params=pltpu.CompilerParams(
            dimension_semantics=("arbitrary",),
        ),
    )(inp, adj, W, a)


# speedup vs baseline: 1.2515x; 1.2515x over previous
import jax
import jax.numpy as jnp
from jax import lax
from jax.experimental import pallas as pl
from jax.experimental.pallas import tpu as pltpu

_B, _N, _INF, _OUTF = 4, 1024, 256, 256
_PB = 2


def _gat_body(inp_ref, adj_ref, w_ref, a_ref, out_ref, at_s):
    @pl.when(pl.program_id(0) == 0)
    def _():
        at_s[...] = jnp.transpose(a_ref[...])

    m = (adj_ref[...] > 0).astype(jnp.float32)
    at = at_s[...]
    row = lax.broadcasted_iota(jnp.int32, (_N, 1), 0)
    for u in range(_PB):
        x = inp_ref[u]
        h = jnp.dot(x, w_ref[...], preferred_element_type=jnp.float32)
        h = jnp.where(row == 0, 0.0, h)
        hp = pltpu.roll(h, 1, 0)
        hp = jnp.where(row == 0, 0.0, hp)
        g = lax.dot_general(m, hp, (((0,), (0,)), ((), ())),
                            preferred_element_type=jnp.float32)
        vals = (jnp.sum(h * at[:, :_OUTF], axis=1, keepdims=True)
                + jnp.sum(g * at[:, _OUTF:], axis=1, keepdims=True))
        vals = jnp.where(row == 0, 0.0, vals)
        ssum = jnp.sum(h, axis=0, keepdims=True)
        o = vals * ssum
        out_ref[u] = jnp.maximum(o, 0.2 * o)


def kernel(inp, adj, W, a):
    return pl.pallas_call(
        _gat_body,
        grid=(_B // _PB,),
        in_specs=[
            pl.BlockSpec((_PB, _N, _INF), lambda b: (b, 0, 0)),
            pl.BlockSpec((_N, _N), lambda b: (0, 0)),
            pl.BlockSpec((_INF, _OUTF), lambda b: (0, 0)),
            pl.BlockSpec((2 * _OUTF, _N), lambda b: (0, 0)),
        ],
        out_specs=pl.BlockSpec((_PB, _N, _OUTF), lambda b: (b, 0, 0)),
        out_shape=jax.ShapeDtypeStruct((_B, _N, _OUTF), jnp.float32),
        scratch_shapes=[pltpu.VMEM((_N, 2 * _OUTF), jnp.float32)],
        compiler_params=pltpu.CompilerParams(
            dimension_semantics=("arbitrary",),
        ),
    )(inp, adj, W, a)
